# baseline (speedup 1.0000x reference)
import functools

import jax
import jax.numpy as jnp
from jax import lax
from jax.experimental import pallas as pl
from jax.experimental.pallas import tpu as pltpu
from jax.experimental.pallas import tpu_sc as plsc

D = 128
B = 4
S = 2048
TOTAL = B * S
NC = 2
NS = 16
L = 16
NW = NC * NS
BPW = TOTAL // NW
CHUNK = 32
NCHUNK = BPW // CHUNK


def _build():
    mesh = plsc.VectorSubcoreMesh(core_axis_name="c", subcore_axis_name="s")

    @functools.partial(
        pl.kernel,
        mesh=mesh,
        out_type=jax.ShapeDtypeStruct((B, S, D), jnp.float32),
        scratch_types=[
            pltpu.VMEM((BPW,), jnp.int32),
            pltpu.VMEM((BPW, D), jnp.float32),
            [pltpu.SemaphoreType.DMA] * NCHUNK,
            [pltpu.SemaphoreType.DMA] * NCHUNK,
            pltpu.SemaphoreType.DMA,
            pltpu.SemaphoreType.DMA,
        ],
    )
    def emb_kernel(idx_hbm, table_hbm, pos_hbm, out_hbm,
                   idx_v, rows_v, gsems, psems, isem, osem):
        c = lax.axis_index("c")
        s = lax.axis_index("s")
        wid = c * NS + s
        gbase = wid * BPW
        b = lax.div(gbase, S)
        off = lax.rem(gbase, S)

        idx_cp = pltpu.async_copy(idx_hbm.at[b, pl.ds(off, BPW)], idx_v, isem)
        pos_cps = [
            pltpu.async_copy(
                pos_hbm.at[pl.ds(off + j * CHUNK, CHUNK)],
                rows_v.at[pl.ds(j * CHUNK, CHUNK)],
                psems[j],
            )
            for j in range(NCHUNK)
        ]
        idx_cp.wait()

        gathers = []
        for j in range(NCHUNK):
            pos_cps[j].wait()
            gathers.append(pltpu.async_copy(
                table_hbm.at[idx_v.at[pl.ds(j * CHUNK, CHUNK)]],
                rows_v.at[pl.ds(j * CHUNK, CHUNK)],
                gsems[j],
                add=True,
            ))
        outs = []
        for j in range(NCHUNK):
            gathers[j].wait()
            base = j * CHUNK
            outs.append(pltpu.async_copy(
                rows_v.at[pl.ds(base, CHUNK)],
                out_hbm.at[b, pl.ds(off + base, CHUNK)],
                osem,
            ))
        for o in outs:
            o.wait()

    return emb_kernel


_emb_kernel = _build()


def kernel(input_ids, token_embeddings, position_embeddings):
    return _emb_kernel(input_ids.astype(jnp.int32), token_embeddings,
                       position_embeddings)


# speedup vs baseline: 1.0176x; 1.0176x over previous
import functools

import jax
import jax.numpy as jnp
from jax import lax
from jax.experimental import pallas as pl
from jax.experimental.pallas import tpu as pltpu
from jax.experimental.pallas import tpu_sc as plsc

D = 128
B = 4
S = 2048
TOTAL = B * S
NC = 2
NS = 16
L = 16
NW = NC * NS
BPW = TOTAL // NW
CHUNKS = (32, 32, 64, 128)
COFFS = (0, 32, 64, 128)
NCHUNK = len(CHUNKS)


def _build():
    mesh = plsc.VectorSubcoreMesh(core_axis_name="c", subcore_axis_name="s")

    @functools.partial(
        pl.kernel,
        mesh=mesh,
        out_type=jax.ShapeDtypeStruct((B, S, D), jnp.float32),
        scratch_types=[
            pltpu.VMEM((BPW,), jnp.int32),
            pltpu.VMEM((BPW, D), jnp.float32),
            [pltpu.SemaphoreType.DMA] * NCHUNK,
            [pltpu.SemaphoreType.DMA] * NCHUNK,
            pltpu.SemaphoreType.DMA,
            pltpu.SemaphoreType.DMA,
        ],
    )
    def emb_kernel(idx_hbm, table_hbm, pos_hbm, out_hbm,
                   idx_v, rows_v, gsems, psems, isem, osem):
        c = lax.axis_index("c")
        s = lax.axis_index("s")
        wid = c * NS + s
        gbase = wid * BPW
        b = lax.div(gbase, S)
        off = lax.rem(gbase, S)

        idx_cp = pltpu.async_copy(idx_hbm.at[b, pl.ds(off, BPW)], idx_v, isem)
        pos_cps = [
            pltpu.async_copy(
                pos_hbm.at[pl.ds(off + COFFS[j], CHUNKS[j])],
                rows_v.at[pl.ds(COFFS[j], CHUNKS[j])],
                psems[j],
            )
            for j in range(NCHUNK)
        ]
        idx_cp.wait()

        gathers = []
        for j in range(NCHUNK):
            pos_cps[j].wait()
            gathers.append(pltpu.async_copy(
                table_hbm.at[idx_v.at[pl.ds(COFFS[j], CHUNKS[j])]],
                rows_v.at[pl.ds(COFFS[j], CHUNKS[j])],
                gsems[j],
                add=True,
            ))
        outs = []
        for j in range(NCHUNK):
            gathers[j].wait()
            outs.append(pltpu.async_copy(
                rows_v.at[pl.ds(COFFS[j], CHUNKS[j])],
                out_hbm.at[b, pl.ds(off + COFFS[j], CHUNKS[j])],
                osem,
            ))
        for o in outs:
            o.wait()

    return emb_kernel


_emb_kernel = _build()


def kernel(input_ids, token_embeddings, position_embeddings):
    return _emb_kernel(input_ids.astype(jnp.int32), token_embeddings,
                       position_embeddings)


# speedup vs baseline: 1.0258x; 1.0080x over previous
import functools

import jax
import jax.numpy as jnp
from jax import lax
from jax.experimental import pallas as pl
from jax.experimental.pallas import tpu as pltpu
from jax.experimental.pallas import tpu_sc as plsc

D = 128
B = 4
S = 2048
TOTAL = B * S
NC = 2
NS = 16
L = 16
NW = NC * NS
BPW = TOTAL // NW
CHUNKS = (16, 16, 32, 64, 128)
COFFS = (0, 16, 32, 64, 128)
NCHUNK = len(CHUNKS)


def _build():
    mesh = plsc.VectorSubcoreMesh(core_axis_name="c", subcore_axis_name="s")

    @functools.partial(
        pl.kernel,
        mesh=mesh,
        out_type=jax.ShapeDtypeStruct((B, S, D), jnp.float32),
        scratch_types=[
            pltpu.VMEM((BPW,), jnp.int32),
            pltpu.VMEM((BPW, D), jnp.float32),
            [pltpu.SemaphoreType.DMA] * NCHUNK,
            [pltpu.SemaphoreType.DMA] * NCHUNK,
            pltpu.SemaphoreType.DMA,
            pltpu.SemaphoreType.DMA,
        ],
    )
    def emb_kernel(idx_hbm, table_hbm, pos_hbm, out_hbm,
                   idx_v, rows_v, gsems, psems, isem, osem):
        c = lax.axis_index("c")
        s = lax.axis_index("s")
        wid = c * NS + s
        gbase = wid * BPW
        b = lax.div(gbase, S)
        off = lax.rem(gbase, S)

        idx_cp = pltpu.async_copy(idx_hbm.at[b, pl.ds(off, BPW)], idx_v, isem)
        pos_cps = [
            pltpu.async_copy(
                pos_hbm.at[pl.ds(off + COFFS[j], CHUNKS[j])],
                rows_v.at[pl.ds(COFFS[j], CHUNKS[j])],
                psems[j],
            )
            for j in range(NCHUNK)
        ]
        idx_cp.wait()

        gathers = []
        for j in range(NCHUNK):
            pos_cps[j].wait()
            gathers.append(pltpu.async_copy(
                table_hbm.at[idx_v.at[pl.ds(COFFS[j], CHUNKS[j])]],
                rows_v.at[pl.ds(COFFS[j], CHUNKS[j])],
                gsems[j],
                add=True,
            ))
        outs = []
        for j in range(NCHUNK):
            gathers[j].wait()
            outs.append(pltpu.async_copy(
                rows_v.at[pl.ds(COFFS[j], CHUNKS[j])],
                out_hbm.at[b, pl.ds(off + COFFS[j], CHUNKS[j])],
                osem,
            ))
        for o in outs:
            o.wait()

    return emb_kernel


_emb_kernel = _build()


def kernel(input_ids, token_embeddings, position_embeddings):
    return _emb_kernel(input_ids.astype(jnp.int32), token_embeddings,
                       position_embeddings)
